# baseline (speedup 1.0000x reference)
import jax
import jax.numpy as jnp
from jax.experimental import pallas as pl
from jax.experimental.pallas import tpu as pltpu

_MIN_NORM = 1e-15
_EPS = 1e-7

_N, _D = 10000, 128
_BM = 200
_NB = _N // _BM
_CB = 8


def _logmap0_kernel(x_ref, o_ref):
    p = x_ref[...]
    p0 = p[:, 0:1]
    y_sq = jnp.sum(p * p, axis=1, keepdims=True) - p0 * p0
    y_norm = jnp.sqrt(jnp.clip(y_sq, _MIN_NORM * _MIN_NORM, None))
    th = jnp.clip(p0, 1.0 + _EPS, None)
    ar = jnp.log(jnp.clip(th + jnp.sqrt(th * th - 1.0), _MIN_NORM, None))
    s = ar / y_norm
    col = jax.lax.broadcasted_iota(jnp.int32, p.shape, 1)
    o_ref[...] = jnp.where(col == 0, 0.0, p * s)


def _expmap0_proj(u):
    u0 = u[:, 0:1]
    x_sq = jnp.sum(u * u, axis=1, keepdims=True) - u0 * u0
    x_norm = jnp.sqrt(jnp.clip(x_sq, _MIN_NORM * _MIN_NORM, None))
    theta = jnp.clip(x_norm, -15.0, 15.0)
    e = jnp.exp(theta)
    sinh = 0.5 * (e - 1.0 / e)
    scale = sinh / x_norm
    y_sq_new = scale * scale * x_sq
    first = jnp.sqrt(jnp.clip(1.0 + y_sq_new, _EPS, None))
    col = jax.lax.broadcasted_iota(jnp.int32, u.shape, 1)
    return jnp.where(col == 0, first, u * scale)


def _fused_kernel(adj_hbm, xt_ref, h_ref, cache, buf, s_ref, s16_ref, sem):
    nb, bm, cb = _NB, _BM, _CB

    def start_dma(i, slot):
        pltpu.make_async_copy(
            adj_hbm.at[pl.ds(i * bm, bm), :], buf.at[slot], sem.at[slot]
        ).start()

    def wait_dma(i, slot):
        pltpu.make_async_copy(
            adj_hbm.at[pl.ds(i * bm, bm), :], buf.at[slot], sem.at[slot]
        ).wait()

    start_dma(0, 0)
    start_dma(1, 1)

    def p0_one(i, slot):
        wait_dma(i, slot)

        @pl.when(i < cb)
        def _():
            cache[i] = buf[slot].astype(jnp.bfloat16)

        acc = jnp.dot(buf[slot], xt_ref[...], preferred_element_type=jnp.float32)
        s_ref[pl.ds(i * bm, bm), :] = acc + xt_ref[pl.ds(i * bm, bm), :]

        @pl.when(i + 2 < nb)
        def _():
            start_dma(i + 2, slot)

    def p0_body(i2, _):
        p0_one(2 * i2, 0)
        p0_one(2 * i2 + 1, 1)
        return 0

    jax.lax.fori_loop(0, nb // 2, p0_body, 0, unroll=False)

    s16_ref[...] = s_ref[...].astype(jnp.bfloat16)
    start_dma(cb, 0)
    start_dma(cb + 1, 1)

    def epilogue(i, acc):
        u = acc + s_ref[pl.ds(i * bm, bm), :] - xt_ref[pl.ds(i * bm, bm), :]
        h_ref[pl.ds(i * bm, bm), :] = _expmap0_proj(u)

    def p1_cached_body(i, _):
        acc = jnp.dot(cache[i], s16_ref[...], preferred_element_type=jnp.float32)
        epilogue(i, acc)
        return 0

    jax.lax.fori_loop(0, cb, p1_cached_body, 0, unroll=False)

    def p1_one(i, slot):
        wait_dma(i, slot)
        acc = jnp.dot(buf[slot], s_ref[...], preferred_element_type=jnp.float32)

        @pl.when(i + 2 < nb)
        def _():
            start_dma(i + 2, slot)

        epilogue(i, acc)

    def p1_body(i2, _):
        p1_one(2 * i2, 0)
        p1_one(2 * i2 + 1, 1)
        return 0

    jax.lax.fori_loop(cb // 2, nb // 2, p1_body, 0, unroll=False)


def kernel(x, adj):
    n, d = x.shape
    bp = 1000
    xt = pl.pallas_call(
        _logmap0_kernel,
        grid=(n // bp,),
        in_specs=[pl.BlockSpec((bp, d), lambda i: (i, 0))],
        out_specs=pl.BlockSpec((bp, d), lambda i: (i, 0)),
        out_shape=jax.ShapeDtypeStruct((n, d), jnp.float32),
    )(x)
    h = pl.pallas_call(
        _fused_kernel,
        in_specs=[
            pl.BlockSpec(memory_space=pl.ANY),
            pl.BlockSpec((n, d), lambda: (0, 0)),
        ],
        out_specs=pl.BlockSpec((n, d), lambda: (0, 0)),
        out_shape=jax.ShapeDtypeStruct((n, d), jnp.float32),
        scratch_shapes=[
            pltpu.VMEM((_CB, _BM, _N), jnp.bfloat16),
            pltpu.VMEM((2, _BM, _N), jnp.float32),
            pltpu.VMEM((_N, _D), jnp.float32),
            pltpu.VMEM((_N, _D), jnp.bfloat16),
            pltpu.SemaphoreType.DMA((2,)),
        ],
        compiler_params=pltpu.CompilerParams(
            vmem_limit_bytes=67000000,
        ),
    )(adj, xt)
    return h


# speedup vs baseline: 1.1235x; 1.1235x over previous
import jax
import jax.numpy as jnp
from jax.experimental import pallas as pl
from jax.experimental.pallas import tpu as pltpu

_MIN_NORM = 1e-15
_EPS = 1e-7

_N, _D = 10000, 128
_BM = 200
_NB = _N // _BM
_CB = 8


def _logmap0_kernel(x_ref, o_ref):
    p = x_ref[...]
    p0 = p[:, 0:1]
    y_sq = jnp.sum(p * p, axis=1, keepdims=True) - p0 * p0
    y_norm = jnp.sqrt(jnp.clip(y_sq, _MIN_NORM * _MIN_NORM, None))
    th = jnp.clip(p0, 1.0 + _EPS, None)
    ar = jnp.log(jnp.clip(th + jnp.sqrt(th * th - 1.0), _MIN_NORM, None))
    s = ar / y_norm
    col = jax.lax.broadcasted_iota(jnp.int32, p.shape, 1)
    o_ref[...] = jnp.where(col == 0, 0.0, p * s)


def _expmap0_proj(u):
    u0 = u[:, 0:1]
    x_sq = jnp.sum(u * u, axis=1, keepdims=True) - u0 * u0
    x_norm = jnp.sqrt(jnp.clip(x_sq, _MIN_NORM * _MIN_NORM, None))
    theta = jnp.clip(x_norm, -15.0, 15.0)
    e = jnp.exp(theta)
    sinh = 0.5 * (e - 1.0 / e)
    scale = sinh / x_norm
    y_sq_new = scale * scale * x_sq
    first = jnp.sqrt(jnp.clip(1.0 + y_sq_new, _EPS, None))
    col = jax.lax.broadcasted_iota(jnp.int32, u.shape, 1)
    return jnp.where(col == 0, first, u * scale)


def _gcn_kernel(adj_ref, xt_ref, hc_ref, hs_ref, cache, s_ref, s16_ref):
    g = pl.program_id(0)
    nb, bm, cb = _NB, _BM, _CB

    @pl.when(g < nb)
    def _():
        i = g

        @pl.when(i < cb)
        def _():
            cache[i] = adj_ref[...].astype(jnp.bfloat16)

        acc = jnp.dot(adj_ref[...], xt_ref[...], preferred_element_type=jnp.float32)
        s_ref[pl.ds(i * bm, bm), :] = acc + xt_ref[pl.ds(i * bm, bm), :]

    @pl.when(g == nb - 1)
    def _():
        s16_ref[...] = s_ref[...].astype(jnp.bfloat16)

    @pl.when(g >= nb)
    def _():
        i = g - nb + cb
        acc = jnp.dot(adj_ref[...], s_ref[...], preferred_element_type=jnp.float32)
        u = acc + s_ref[pl.ds(i * bm, bm), :] - xt_ref[pl.ds(i * bm, bm), :]
        hs_ref[...] = _expmap0_proj(u)

        j = g - nb

        @pl.when(j < cb)
        def _():
            acc2 = jnp.dot(cache[j], s16_ref[...], preferred_element_type=jnp.float32)
            u2 = acc2 + s_ref[pl.ds(j * bm, bm), :] - xt_ref[pl.ds(j * bm, bm), :]
            hc_ref[...] = _expmap0_proj(u2)


def kernel(x, adj):
    n, d = x.shape
    nb, bm, cb = _NB, _BM, _CB
    bp = 1000
    xt = pl.pallas_call(
        _logmap0_kernel,
        grid=(n // bp,),
        in_specs=[pl.BlockSpec((bp, d), lambda i: (i, 0))],
        out_specs=pl.BlockSpec((bp, d), lambda i: (i, 0)),
        out_shape=jax.ShapeDtypeStruct((n, d), jnp.float32),
    )(x)
    hc, hs = pl.pallas_call(
        _gcn_kernel,
        grid=(2 * nb - cb,),
        in_specs=[
            pl.BlockSpec((bm, n), lambda g: (jnp.where(g < nb, g, g - nb + cb), 0)),
            pl.BlockSpec((n, d), lambda g: (0, 0)),
        ],
        out_specs=[
            pl.BlockSpec(
                (bm, d),
                lambda g: (jnp.minimum(jnp.maximum(g - nb, 0), cb - 1), 0),
            ),
            pl.BlockSpec((bm, d), lambda g: (jnp.where(g < nb, 0, g - nb), 0)),
        ],
        out_shape=[
            jax.ShapeDtypeStruct((cb * bm, d), jnp.float32),
            jax.ShapeDtypeStruct((n - cb * bm, d), jnp.float32),
        ],
        scratch_shapes=[
            pltpu.VMEM((_CB, _BM, _N), jnp.bfloat16),
            pltpu.VMEM((_N, _D), jnp.float32),
            pltpu.VMEM((_N, _D), jnp.bfloat16),
        ],
        compiler_params=pltpu.CompilerParams(
            dimension_semantics=("arbitrary",),
            vmem_limit_bytes=67000000,
        ),
    )(adj, xt)
    return jnp.concatenate([hc, hs], axis=0)
